# baseline (speedup 1.0000x reference)
import functools

import jax
import jax.numpy as jnp
from jax.experimental import pallas as pl
from jax.experimental.pallas import tpu as pltpu

_N = 10000
_D = 128


def _gcn_block_kernel(adj_ref, xv_ref, bias_ref, o_ref, *, blk):
    i = pl.program_id(0)
    o_ref[...] = (
        jnp.dot(adj_ref[...], xv_ref[...], preferred_element_type=jnp.float32)
        + bias_ref[pl.ds(i * blk, blk), :]
    )


@jax.jit
def kernel(x, adj, h0, W, lamda, alpha, layer_idx):
    n, d = x.shape
    blk = 400 if n % 400 == 0 else 8
    theta = jnp.log(
        jnp.asarray(lamda, jnp.float32) / jnp.asarray(layer_idx, jnp.float32) + 1.0
    )
    alpha = jnp.asarray(alpha, jnp.float32)
    v = theta * W + (1.0 - theta) * jnp.eye(d, dtype=jnp.float32)
    xv = (1.0 - alpha) * (x @ v)
    bias = alpha * (h0 @ v) + x

    grid = (n // blk,)
    out = pl.pallas_call(
        functools.partial(_gcn_block_kernel, blk=blk),
        grid=grid,
        in_specs=[
            pl.BlockSpec((blk, n), lambda i: (i, 0)),
            pl.BlockSpec((n, d), lambda i: (0, 0)),
            pl.BlockSpec((n, d), lambda i: (0, 0)),
        ],
        out_specs=pl.BlockSpec((blk, d), lambda i: (i, 0)),
        out_shape=jax.ShapeDtypeStruct((n, d), jnp.float32),
        compiler_params=pltpu.CompilerParams(
            dimension_semantics=("arbitrary",),
        ),
    )(adj, xv, bias)
    return out


# speedup vs baseline: 1.0653x; 1.0653x over previous
import functools

import jax
import jax.numpy as jnp
from jax.experimental import pallas as pl
from jax.experimental.pallas import tpu as pltpu

_N = 10000
_D = 128


def _gcn_block_kernel(scal_ref, adj_ref, x_ref, h0_ref, w_ref, o_ref, *, blk):
    theta = scal_ref[0]
    alpha = scal_ref[1]
    i = pl.program_id(0)
    hi = jnp.dot(adj_ref[...], x_ref[...], preferred_element_type=jnp.float32)
    h0_blk = h0_ref[pl.ds(i * blk, blk), :]
    support = (1.0 - alpha) * hi + alpha * h0_blk
    x_blk = x_ref[pl.ds(i * blk, blk), :]
    o_ref[...] = (
        theta * jnp.dot(support, w_ref[...], preferred_element_type=jnp.float32)
        + (1.0 - theta) * support
        + x_blk
    )


@functools.partial(jax.jit, static_argnames=())
def kernel(x, adj, h0, W, lamda, alpha, layer_idx):
    n, d = x.shape
    blk = 400 if n % 400 == 0 else 8
    theta = jnp.log(
        jnp.asarray(lamda, jnp.float32) / jnp.asarray(layer_idx, jnp.float32) + 1.0
    )
    scal = jnp.stack([theta, jnp.asarray(alpha, jnp.float32)])

    grid = (n // blk,)
    out = pl.pallas_call(
        functools.partial(_gcn_block_kernel, blk=blk),
        grid=grid,
        in_specs=[
            pl.BlockSpec(memory_space=pltpu.SMEM),
            pl.BlockSpec((blk, n), lambda i: (i, 0)),
            pl.BlockSpec((n, d), lambda i: (0, 0)),
            pl.BlockSpec((n, d), lambda i: (0, 0)),
            pl.BlockSpec((d, d), lambda i: (0, 0)),
        ],
        out_specs=pl.BlockSpec((blk, d), lambda i: (i, 0)),
        out_shape=jax.ShapeDtypeStruct((n, d), jnp.float32),
        compiler_params=pltpu.CompilerParams(
            dimension_semantics=("arbitrary",),
        ),
    )(scal, adj, x, h0, W)
    return out
